# Initial kernel scaffold; baseline (speedup 1.0000x reference)
import functools

import jax
import jax.numpy as jnp
from jax import lax
from jax.experimental import pallas as pl
from jax.experimental.pallas import tpu as pltpu
from jax.experimental.pallas import tpu_sc as plsc

N = 10000
E = 320000
D = 128

NC = 2
NS = 16
NW = NC * NS
PER_W = E // NW
K = 80
NB = PER_W // K
RPS = N // NS
ZB = 125

_mesh = plsc.VectorSubcoreMesh(core_axis_name="c", subcore_axis_name="s")


def _sc_deg(dst):
    @functools.partial(
        pl.kernel,
        mesh=_mesh,
        out_type=jax.ShapeDtypeStruct((NC, N, 16), jnp.float32),
        scratch_types=[
            pltpu.VMEM((K,), jnp.int32),
            pltpu.VMEM((ZB, 16), jnp.float32),
            pltpu.VMEM((K, 16), jnp.float32),
            pltpu.VMEM_SHARED((N, 16), jnp.float32),
        ],
    )
    def k(dst_hbm, out_hbm, idx_v, zb_v, ones_v, acc_sh):
        cid = lax.axis_index("c")
        sid = lax.axis_index("s")
        wid = sid * NC + cid

        @pl.loop(0, ZB)
        def _(r):
            zb_v[r, pl.ds(0, 16)] = jnp.zeros((16,), jnp.float32)

        @pl.loop(0, K)
        def _(r):
            ones_v[r, pl.ds(0, 16)] = jnp.ones((16,), jnp.float32)

        @pl.loop(0, RPS // ZB)
        def _(t):
            pltpu.sync_copy(zb_v, acc_sh.at[pl.ds(sid * RPS + t * ZB, ZB)])

        plsc.subcore_barrier()

        base = wid * PER_W

        @pl.loop(0, NB)
        def _(j):
            pltpu.sync_copy(dst_hbm.at[pl.ds(base + j * K, K)], idx_v)
            pltpu.sync_copy(ones_v, acc_sh.at[idx_v], add=True)

        plsc.subcore_barrier()
        pltpu.sync_copy(
            acc_sh.at[pl.ds(sid * RPS, RPS)],
            out_hbm.at[cid, pl.ds(sid * RPS, RPS)],
        )

    return k(dst)


def _sc_agg(hp, src, dst):
    @functools.partial(
        pl.kernel,
        mesh=_mesh,
        out_type=jax.ShapeDtypeStruct((NC, N, D), jnp.float32),
        scratch_types=[
            pltpu.VMEM((K,), jnp.int32),
            pltpu.VMEM((K,), jnp.int32),
            pltpu.VMEM((K, D), jnp.float32),
            pltpu.VMEM((ZB, D), jnp.float32),
            pltpu.SemaphoreType.DMA,
            pltpu.VMEM_SHARED((N, D), jnp.float32),
        ],
    )
    def k(hp_hbm, src_hbm, dst_hbm, out_hbm, src_v, dst_v, rows_v, zb_v, sem, acc_sh):
        cid = lax.axis_index("c")
        sid = lax.axis_index("s")
        wid = sid * NC + cid

        @pl.loop(0, ZB)
        def _(r):
            @pl.loop(0, D, step=16)
            def _(c):
                zb_v[r, pl.ds(c, 16)] = jnp.zeros((16,), jnp.float32)

        @pl.loop(0, RPS // ZB)
        def _(t):
            pltpu.sync_copy(zb_v, acc_sh.at[pl.ds(sid * RPS + t * ZB, ZB)])

        plsc.subcore_barrier()

        base = wid * PER_W

        @pl.loop(0, NB)
        def _(j):
            pltpu.sync_copy(src_hbm.at[pl.ds(base + j * K, K)], src_v)
            pltpu.sync_copy(dst_hbm.at[pl.ds(base + j * K, K)], dst_v)
            pltpu.async_copy(hp_hbm.at[src_v], rows_v, sem).wait()
            pltpu.sync_copy(rows_v, acc_sh.at[dst_v], add=True)

        plsc.subcore_barrier()
        pltpu.sync_copy(
            acc_sh.at[pl.ds(sid * RPS, RPS)],
            out_hbm.at[cid, pl.ds(sid * RPS, RPS)],
        )

    return k(hp, src, dst)


_R = 1000


def _mm_body(x_ref, w_ref, o_ref):
    o_ref[...] = jnp.dot(x_ref[...], w_ref[...], preferred_element_type=jnp.float32)


def _tc_mm(x, w):
    return pl.pallas_call(
        _mm_body,
        grid=(N // _R,),
        in_specs=[
            pl.BlockSpec((_R, x.shape[1]), lambda i: (i, 0)),
            pl.BlockSpec(w.shape, lambda i: (0, 0)),
        ],
        out_specs=pl.BlockSpec((_R, w.shape[1]), lambda i: (i, 0)),
        out_shape=jax.ShapeDtypeStruct((N, w.shape[1]), jnp.float32),
    )(x, w)


def _scale_body(degp_ref, h_ref, hp_ref, dinv_ref):
    d = degp_ref[0, :, 0:1] + degp_ref[1, :, 0:1] + 1.0
    dinv = lax.rsqrt(d)
    dinv_ref[...] = dinv
    hp_ref[...] = h_ref[...] * dinv


def _tc_scale(degp, h):
    return pl.pallas_call(
        _scale_body,
        grid=(N // _R,),
        in_specs=[
            pl.BlockSpec((NC, _R, 16), lambda i: (0, i, 0)),
            pl.BlockSpec((_R, D), lambda i: (i, 0)),
        ],
        out_specs=[
            pl.BlockSpec((_R, D), lambda i: (i, 0)),
            pl.BlockSpec((_R, 1), lambda i: (i, 0)),
        ],
        out_shape=[
            jax.ShapeDtypeStruct((N, D), jnp.float32),
            jax.ShapeDtypeStruct((N, 1), jnp.float32),
        ],
    )(degp, h)


def _mid_body(acc_ref, hp_ref, dinv_ref, b_ref, w_ref, o_ref):
    dinv = dinv_ref[...]
    z = dinv * (acc_ref[0] + acc_ref[1] + hp_ref[...]) + b_ref[...]
    z = jnp.maximum(z, 0.0)
    o_ref[...] = jnp.dot(z, w_ref[...], preferred_element_type=jnp.float32) * dinv


def _tc_mid(acc, hp, dinv, b, w):
    return pl.pallas_call(
        _mid_body,
        grid=(N // _R,),
        in_specs=[
            pl.BlockSpec((NC, _R, D), lambda i: (0, i, 0)),
            pl.BlockSpec((_R, D), lambda i: (i, 0)),
            pl.BlockSpec((_R, 1), lambda i: (i, 0)),
            pl.BlockSpec((1, D), lambda i: (0, 0)),
            pl.BlockSpec((D, D), lambda i: (0, 0)),
        ],
        out_specs=pl.BlockSpec((_R, D), lambda i: (i, 0)),
        out_shape=jax.ShapeDtypeStruct((N, D), jnp.float32),
    )(acc, hp, dinv, b, w)


def _final_body(acc_ref, hp_ref, dinv_ref, b_ref, wo_ref, bo_ref, emb_ref, out_ref):
    emb = dinv_ref[...] * (acc_ref[0] + acc_ref[1] + hp_ref[...]) + b_ref[...]
    emb_ref[...] = emb
    out_ref[...] = (
        jnp.dot(emb, wo_ref[...], preferred_element_type=jnp.float32) + bo_ref[...]
    )


def _tc_final(acc, hp, dinv, b, wo_t, bo):
    d_out = wo_t.shape[1]
    return pl.pallas_call(
        _final_body,
        grid=(N // _R,),
        in_specs=[
            pl.BlockSpec((NC, _R, D), lambda i: (0, i, 0)),
            pl.BlockSpec((_R, D), lambda i: (i, 0)),
            pl.BlockSpec((_R, 1), lambda i: (i, 0)),
            pl.BlockSpec((1, D), lambda i: (0, 0)),
            pl.BlockSpec((D, d_out), lambda i: (0, 0)),
            pl.BlockSpec((1, d_out), lambda i: (0, 0)),
        ],
        out_specs=[
            pl.BlockSpec((_R, D), lambda i: (i, 0)),
            pl.BlockSpec((_R, d_out), lambda i: (i, 0)),
        ],
        out_shape=[
            jax.ShapeDtypeStruct((N, D), jnp.float32),
            jax.ShapeDtypeStruct((N, d_out), jnp.float32),
        ],
    )(acc, hp, dinv, b, wo_t, bo)


@jax.jit
def kernel(x, edge_index, W1, b1, W2, b2, W_out, b_out):
    src = edge_index[0]
    dst = edge_index[1]

    degp = _sc_deg(dst)
    h1 = _tc_mm(x, W1)
    h1p, dinv = _tc_scale(degp, h1)

    acc1 = _sc_agg(h1p, src, dst)
    h2p = _tc_mid(acc1, h1p, dinv, b1.reshape(1, D), W2)

    acc2 = _sc_agg(h2p, src, dst)
    emb, out = _tc_final(
        acc2, h2p, dinv, b2.reshape(1, D), W_out.T, b_out.reshape(1, -1)
    )
    return (out, emb)


# speedup vs baseline: 12.3586x; 12.3586x over previous
import functools

import jax
import jax.numpy as jnp
from jax import lax
from jax.experimental import pallas as pl
from jax.experimental.pallas import tpu as pltpu
from jax.experimental.pallas import tpu_sc as plsc

N = 10000
N_PAD = 10240
E = 320000
D = 128

NC = 2
NS = 16
NW = NC * NS
PER_W = E // NW
K = 80
NB = PER_W // K
RPS = N_PAD // NS
ZB = 128

_mesh = plsc.VectorSubcoreMesh(core_axis_name="c", subcore_axis_name="s")


def _sc_deg(dst):
    @functools.partial(
        pl.kernel,
        mesh=_mesh,
        out_type=jax.ShapeDtypeStruct((NC, N_PAD, D), jnp.float32),
        scratch_types=[
            pltpu.VMEM((K,), jnp.int32),
            pltpu.VMEM((ZB, D), jnp.float32),
            pltpu.VMEM((K, D), jnp.float32),
            pltpu.VMEM_SHARED((N_PAD, D), jnp.float32),
        ],
    )
    def k(dst_hbm, out_hbm, idx_v, zb_v, ones_v, acc_sh):
        cid = lax.axis_index("c")
        sid = lax.axis_index("s")
        wid = sid * NC + cid

        @pl.loop(0, ZB)
        def _(r):
            @pl.loop(0, D, step=16)
            def _(c):
                zb_v[r, pl.ds(c, 16)] = jnp.zeros((16,), jnp.float32)

        @pl.loop(0, K)
        def _(r):
            @pl.loop(0, D, step=16)
            def _(c):
                ones_v[r, pl.ds(c, 16)] = jnp.ones((16,), jnp.float32)

        @pl.loop(0, RPS // ZB)
        def _(t):
            pltpu.sync_copy(zb_v, acc_sh.at[pl.ds(sid * RPS + t * ZB, ZB)])

        plsc.subcore_barrier()

        base = wid * PER_W

        @pl.loop(0, NB)
        def _(j):
            pltpu.sync_copy(dst_hbm.at[pl.ds(base + j * K, K)], idx_v)
            pltpu.sync_copy(ones_v, acc_sh.at[idx_v], add=True)

        plsc.subcore_barrier()
        pltpu.sync_copy(
            acc_sh.at[pl.ds(sid * RPS, RPS)],
            out_hbm.at[cid, pl.ds(sid * RPS, RPS)],
        )

    return k(dst)


def _sc_agg(hp, src, dst):
    @functools.partial(
        pl.kernel,
        mesh=_mesh,
        out_type=jax.ShapeDtypeStruct((NC, N_PAD, D), jnp.float32),
        scratch_types=[
            pltpu.VMEM((K,), jnp.int32),
            pltpu.VMEM((K,), jnp.int32),
            pltpu.VMEM((K, D), jnp.float32),
            pltpu.VMEM((ZB, D), jnp.float32),
            pltpu.SemaphoreType.DMA,
            pltpu.VMEM_SHARED((N_PAD, D), jnp.float32),
        ],
    )
    def k(hp_hbm, src_hbm, dst_hbm, out_hbm, src_v, dst_v, rows_v, zb_v, sem, acc_sh):
        cid = lax.axis_index("c")
        sid = lax.axis_index("s")
        wid = sid * NC + cid

        @pl.loop(0, ZB)
        def _(r):
            @pl.loop(0, D, step=16)
            def _(c):
                zb_v[r, pl.ds(c, 16)] = jnp.zeros((16,), jnp.float32)

        @pl.loop(0, RPS // ZB)
        def _(t):
            pltpu.sync_copy(zb_v, acc_sh.at[pl.ds(sid * RPS + t * ZB, ZB)])

        plsc.subcore_barrier()

        base = wid * PER_W

        @pl.loop(0, NB)
        def _(j):
            pltpu.sync_copy(src_hbm.at[pl.ds(base + j * K, K)], src_v)
            pltpu.sync_copy(dst_hbm.at[pl.ds(base + j * K, K)], dst_v)
            pltpu.async_copy(hp_hbm.at[src_v], rows_v, sem).wait()
            pltpu.sync_copy(rows_v, acc_sh.at[dst_v], add=True)

        plsc.subcore_barrier()
        pltpu.sync_copy(
            acc_sh.at[pl.ds(sid * RPS, RPS)],
            out_hbm.at[cid, pl.ds(sid * RPS, RPS)],
        )

    return k(hp, src, dst)


_R = 1000


def _mm_body(x_ref, w_ref, o_ref):
    o_ref[...] = jnp.dot(x_ref[...], w_ref[...], preferred_element_type=jnp.float32)


def _tc_mm(x, w):
    return pl.pallas_call(
        _mm_body,
        grid=(N // _R,),
        in_specs=[
            pl.BlockSpec((_R, x.shape[1]), lambda i: (i, 0)),
            pl.BlockSpec(w.shape, lambda i: (0, 0)),
        ],
        out_specs=pl.BlockSpec((_R, w.shape[1]), lambda i: (i, 0)),
        out_shape=jax.ShapeDtypeStruct((N, w.shape[1]), jnp.float32),
    )(x, w)


def _scale_body(degp_ref, h_ref, hp_ref, dinv_ref):
    d = degp_ref[0, :, 0:1] + degp_ref[1, :, 0:1] + 1.0
    dinv = lax.rsqrt(d)
    dinv_ref[...] = dinv
    hp_ref[...] = h_ref[...] * dinv


def _tc_scale(degp, h):
    return pl.pallas_call(
        _scale_body,
        grid=(N // _R,),
        in_specs=[
            pl.BlockSpec((NC, _R, D), lambda i: (0, i, 0)),
            pl.BlockSpec((_R, D), lambda i: (i, 0)),
        ],
        out_specs=[
            pl.BlockSpec((_R, D), lambda i: (i, 0)),
            pl.BlockSpec((_R, 1), lambda i: (i, 0)),
        ],
        out_shape=[
            jax.ShapeDtypeStruct((N, D), jnp.float32),
            jax.ShapeDtypeStruct((N, 1), jnp.float32),
        ],
    )(degp, h)


def _mid_body(acc_ref, hp_ref, dinv_ref, b_ref, w_ref, o_ref):
    dinv = dinv_ref[...]
    z = dinv * (acc_ref[0] + acc_ref[1] + hp_ref[...]) + b_ref[...]
    z = jnp.maximum(z, 0.0)
    o_ref[...] = jnp.dot(z, w_ref[...], preferred_element_type=jnp.float32) * dinv


def _tc_mid(acc, hp, dinv, b, w):
    return pl.pallas_call(
        _mid_body,
        grid=(N // _R,),
        in_specs=[
            pl.BlockSpec((NC, _R, D), lambda i: (0, i, 0)),
            pl.BlockSpec((_R, D), lambda i: (i, 0)),
            pl.BlockSpec((_R, 1), lambda i: (i, 0)),
            pl.BlockSpec((1, D), lambda i: (0, 0)),
            pl.BlockSpec((D, D), lambda i: (0, 0)),
        ],
        out_specs=pl.BlockSpec((_R, D), lambda i: (i, 0)),
        out_shape=jax.ShapeDtypeStruct((N, D), jnp.float32),
    )(acc, hp, dinv, b, w)


def _final_body(acc_ref, hp_ref, dinv_ref, b_ref, wo_ref, bo_ref, emb_ref, out_ref):
    emb = dinv_ref[...] * (acc_ref[0] + acc_ref[1] + hp_ref[...]) + b_ref[...]
    emb_ref[...] = emb
    out_ref[...] = (
        jnp.dot(emb, wo_ref[...], preferred_element_type=jnp.float32) + bo_ref[...]
    )


def _tc_final(acc, hp, dinv, b, wo_t, bo):
    d_out = wo_t.shape[1]
    return pl.pallas_call(
        _final_body,
        grid=(N // _R,),
        in_specs=[
            pl.BlockSpec((NC, _R, D), lambda i: (0, i, 0)),
            pl.BlockSpec((_R, D), lambda i: (i, 0)),
            pl.BlockSpec((_R, 1), lambda i: (i, 0)),
            pl.BlockSpec((1, D), lambda i: (0, 0)),
            pl.BlockSpec((D, d_out), lambda i: (0, 0)),
            pl.BlockSpec((1, d_out), lambda i: (0, 0)),
        ],
        out_specs=[
            pl.BlockSpec((_R, D), lambda i: (i, 0)),
            pl.BlockSpec((_R, d_out), lambda i: (i, 0)),
        ],
        out_shape=[
            jax.ShapeDtypeStruct((N, D), jnp.float32),
            jax.ShapeDtypeStruct((N, d_out), jnp.float32),
        ],
    )(acc, hp, dinv, b, wo_t, bo)


@jax.jit
def kernel(x, edge_index, W1, b1, W2, b2, W_out, b_out):
    src = edge_index[0]
    dst = edge_index[1]

    degp = _sc_deg(dst)
    h1 = _tc_mm(x, W1)
    h1p, dinv = _tc_scale(degp, h1)

    acc1 = _sc_agg(h1p, src, dst)
    h2p = _tc_mid(acc1, h1p, dinv, b1.reshape(1, D), W2)

    acc2 = _sc_agg(h2p, src, dst)
    emb, out = _tc_final(
        acc2, h2p, dinv, b2.reshape(1, D), W_out.T, b_out.reshape(1, -1)
    )
    return (out, emb)


# speedup vs baseline: 22.1270x; 1.7904x over previous
import functools

import jax
import jax.numpy as jnp
from jax import lax
from jax.experimental import pallas as pl
from jax.experimental.pallas import tpu as pltpu
from jax.experimental.pallas import tpu_sc as plsc

N = 10000
N_PAD = 10240
E = 320000
D = 128

NC = 2
NS = 16
NW = NC * NS
PER_W = E // NW
K = 80
NB = PER_W // K
RPS = N_PAD // NS
ZB = 32

_mesh = plsc.VectorSubcoreMesh(core_axis_name="c", subcore_axis_name="s")


def _sc_deg(dst3):
    @functools.partial(
        pl.kernel,
        mesh=_mesh,
        out_type=jax.ShapeDtypeStruct((NC, N_PAD, D), jnp.float32),
        scratch_types=[
            pltpu.VMEM((NB, K), jnp.int32),
            pltpu.VMEM((ZB, D), jnp.float32),
            pltpu.VMEM((K, D), jnp.float32),
            pltpu.SemaphoreType.DMA,
            pltpu.VMEM_SHARED((N_PAD, D), jnp.float32),
        ],
    )
    def k(dst_hbm, out_hbm, idx_v, zb_v, ones_v, ssem, acc_sh):
        cid = lax.axis_index("c")
        sid = lax.axis_index("s")
        wid = sid * NC + cid

        @pl.loop(0, ZB)
        def _(r):
            @pl.loop(0, D, step=16)
            def _(c):
                zb_v[r, pl.ds(c, 16)] = jnp.zeros((16,), jnp.float32)

        @pl.loop(0, K)
        def _(r):
            @pl.loop(0, D, step=16)
            def _(c):
                ones_v[r, pl.ds(c, 16)] = jnp.ones((16,), jnp.float32)

        @pl.loop(0, RPS // ZB)
        def _(t):
            pltpu.sync_copy(zb_v, acc_sh.at[pl.ds(sid * RPS + t * ZB, ZB)])

        pltpu.sync_copy(dst_hbm.at[wid], idx_v)
        plsc.subcore_barrier()

        @pl.loop(0, NB)
        def _(j):
            pltpu.async_copy(ones_v, acc_sh.at[idx_v.at[j]], ssem, add=True)

        @pl.loop(0, NB)
        def _(j):
            pltpu.make_async_copy(ones_v, acc_sh.at[idx_v.at[j]], ssem).wait()

        plsc.subcore_barrier()
        pltpu.sync_copy(
            acc_sh.at[pl.ds(sid * RPS, RPS)],
            out_hbm.at[cid, pl.ds(sid * RPS, RPS)],
        )

    return k(dst3)


def _sc_agg(hp, src2, dst3):
    @functools.partial(
        pl.kernel,
        mesh=_mesh,
        out_type=jax.ShapeDtypeStruct((NC, N_PAD, D), jnp.float32),
        scratch_types=[
            pltpu.VMEM((PER_W,), jnp.int32),
            pltpu.VMEM((1, K), jnp.int32),
            pltpu.VMEM((1, K), jnp.int32),
            pltpu.VMEM((K, D), jnp.float32),
            pltpu.VMEM((K, D), jnp.float32),
            pltpu.VMEM((ZB, D), jnp.float32),
            pltpu.SemaphoreType.DMA,
            pltpu.SemaphoreType.DMA,
            pltpu.SemaphoreType.DMA,
            pltpu.VMEM_SHARED((N_PAD, D), jnp.float32),
        ],
    )
    def k(hp_hbm, src_hbm, dst_hbm, out_hbm,
          src_v, dst0, dst1, rows0, rows1, zb_v, gsem, dsem0, dsem1, acc_sh):
        cid = lax.axis_index("c")
        sid = lax.axis_index("s")
        wid = sid * NC + cid

        @pl.loop(0, ZB)
        def _(r):
            @pl.loop(0, D, step=16)
            def _(c):
                zb_v[r, pl.ds(c, 16)] = jnp.zeros((16,), jnp.float32)

        @pl.loop(0, RPS // ZB)
        def _(t):
            pltpu.sync_copy(zb_v, acc_sh.at[pl.ds(sid * RPS + t * ZB, ZB)])

        pltpu.sync_copy(src_hbm.at[wid], src_v)
        plsc.subcore_barrier()

        def gidx(j):
            return src_v.at[pl.ds(j * K, K)]

        def dblk(j):
            return dst_hbm.at[wid, pl.ds(j, 1)]

        pltpu.async_copy(dblk(0), dst0, dsem0)
        pltpu.async_copy(dblk(1), dst1, dsem1)
        pltpu.async_copy(hp_hbm.at[gidx(0)], rows0, gsem)

        @pl.loop(0, NB - 1, step=2)
        def _(j):
            pltpu.make_async_copy(hp_hbm.at[gidx(j)], rows0, gsem).wait()
            pltpu.async_copy(hp_hbm.at[gidx(j + 1)], rows1, gsem)
            pltpu.make_async_copy(dblk(j), dst0, dsem0).wait()
            pltpu.sync_copy(rows0, acc_sh.at[dst0.at[0]], add=True)
            pltpu.async_copy(dblk(j + 2), dst0, dsem0)
            pltpu.make_async_copy(hp_hbm.at[gidx(j + 1)], rows1, gsem).wait()
            pltpu.async_copy(hp_hbm.at[gidx(j + 2)], rows0, gsem)
            pltpu.make_async_copy(dblk(j + 1), dst1, dsem1).wait()
            pltpu.sync_copy(rows1, acc_sh.at[dst1.at[0]], add=True)

            @pl.when(j + 3 < NB)
            def _():
                pltpu.async_copy(dblk(j + 3), dst1, dsem1)

        pltpu.make_async_copy(hp_hbm.at[gidx(NB - 1)], rows0, gsem).wait()
        pltpu.make_async_copy(dblk(NB - 1), dst0, dsem0).wait()
        pltpu.sync_copy(rows0, acc_sh.at[dst0.at[0]], add=True)

        plsc.subcore_barrier()
        pltpu.sync_copy(
            acc_sh.at[pl.ds(sid * RPS, RPS)],
            out_hbm.at[cid, pl.ds(sid * RPS, RPS)],
        )

    return k(hp, src2, dst3)


_R = 1000


def _mm_body(x_ref, w_ref, o_ref):
    o_ref[...] = jnp.dot(x_ref[...], w_ref[...], preferred_element_type=jnp.float32)


def _tc_mm(x, w):
    return pl.pallas_call(
        _mm_body,
        grid=(N // _R,),
        in_specs=[
            pl.BlockSpec((_R, x.shape[1]), lambda i: (i, 0)),
            pl.BlockSpec(w.shape, lambda i: (0, 0)),
        ],
        out_specs=pl.BlockSpec((_R, w.shape[1]), lambda i: (i, 0)),
        out_shape=jax.ShapeDtypeStruct((N, w.shape[1]), jnp.float32),
    )(x, w)


def _scale_body(degp_ref, h_ref, hp_ref, dinv_ref):
    d = degp_ref[0, :, 0:1] + degp_ref[1, :, 0:1] + 1.0
    dinv = lax.rsqrt(d)
    dinv_ref[...] = dinv
    hp_ref[...] = h_ref[...] * dinv


def _tc_scale(degp, h):
    return pl.pallas_call(
        _scale_body,
        grid=(N // _R,),
        in_specs=[
            pl.BlockSpec((NC, _R, D), lambda i: (0, i, 0)),
            pl.BlockSpec((_R, D), lambda i: (i, 0)),
        ],
        out_specs=[
            pl.BlockSpec((_R, D), lambda i: (i, 0)),
            pl.BlockSpec((_R, 1), lambda i: (i, 0)),
        ],
        out_shape=[
            jax.ShapeDtypeStruct((N, D), jnp.float32),
            jax.ShapeDtypeStruct((N, 1), jnp.float32),
        ],
    )(degp, h)


def _mid_body(acc_ref, hp_ref, dinv_ref, b_ref, w_ref, o_ref):
    dinv = dinv_ref[...]
    z = dinv * (acc_ref[0] + acc_ref[1] + hp_ref[...]) + b_ref[...]
    z = jnp.maximum(z, 0.0)
    o_ref[...] = jnp.dot(z, w_ref[...], preferred_element_type=jnp.float32) * dinv


def _tc_mid(acc, hp, dinv, b, w):
    return pl.pallas_call(
        _mid_body,
        grid=(N // _R,),
        in_specs=[
            pl.BlockSpec((NC, _R, D), lambda i: (0, i, 0)),
            pl.BlockSpec((_R, D), lambda i: (i, 0)),
            pl.BlockSpec((_R, 1), lambda i: (i, 0)),
            pl.BlockSpec((1, D), lambda i: (0, 0)),
            pl.BlockSpec((D, D), lambda i: (0, 0)),
        ],
        out_specs=pl.BlockSpec((_R, D), lambda i: (i, 0)),
        out_shape=jax.ShapeDtypeStruct((N, D), jnp.float32),
    )(acc, hp, dinv, b, w)


def _final_body(acc_ref, hp_ref, dinv_ref, b_ref, wo_ref, bo_ref, emb_ref, out_ref):
    emb = dinv_ref[...] * (acc_ref[0] + acc_ref[1] + hp_ref[...]) + b_ref[...]
    emb_ref[...] = emb
    out_ref[...] = (
        jnp.dot(emb, wo_ref[...], preferred_element_type=jnp.float32) + bo_ref[...]
    )


def _tc_final(acc, hp, dinv, b, wo_t, bo):
    d_out = wo_t.shape[1]
    return pl.pallas_call(
        _final_body,
        grid=(N // _R,),
        in_specs=[
            pl.BlockSpec((NC, _R, D), lambda i: (0, i, 0)),
            pl.BlockSpec((_R, D), lambda i: (i, 0)),
            pl.BlockSpec((_R, 1), lambda i: (i, 0)),
            pl.BlockSpec((1, D), lambda i: (0, 0)),
            pl.BlockSpec((D, d_out), lambda i: (0, 0)),
            pl.BlockSpec((1, d_out), lambda i: (0, 0)),
        ],
        out_specs=[
            pl.BlockSpec((_R, D), lambda i: (i, 0)),
            pl.BlockSpec((_R, d_out), lambda i: (i, 0)),
        ],
        out_shape=[
            jax.ShapeDtypeStruct((N, D), jnp.float32),
            jax.ShapeDtypeStruct((N, d_out), jnp.float32),
        ],
    )(acc, hp, dinv, b, wo_t, bo)


@jax.jit
def kernel(x, edge_index, W1, b1, W2, b2, W_out, b_out):
    src2 = edge_index[0].reshape(NW, PER_W)
    dst3 = edge_index[1].reshape(NW, NB, K)

    degp = _sc_deg(dst3)
    h1 = _tc_mm(x, W1)
    h1p, dinv = _tc_scale(degp, h1)

    acc1 = _sc_agg(h1p, src2, dst3)
    h2p = _tc_mid(acc1, h1p, dinv, b1.reshape(1, D), W2)

    acc2 = _sc_agg(h2p, src2, dst3)
    emb, out = _tc_final(
        acc2, h2p, dinv, b2.reshape(1, D), W_out.T, b_out.reshape(1, -1)
    )
    return (out, emb)


# speedup vs baseline: 29.5859x; 1.3371x over previous
import functools

import jax
import jax.numpy as jnp
from jax import lax
from jax.experimental import pallas as pl
from jax.experimental.pallas import tpu as pltpu
from jax.experimental.pallas import tpu_sc as plsc

N = 10000
N_PAD = 10240
E = 320000
D = 128

NC = 2
NS = 16
NW = NC * NS
PER_W = E // NW
K = 80
NB = PER_W // K
RPS = N_PAD // NS
ZB = 32

_mesh = plsc.VectorSubcoreMesh(core_axis_name="c", subcore_axis_name="s")


def _sc_deg(dst3):
    @functools.partial(
        pl.kernel,
        mesh=_mesh,
        out_type=jax.ShapeDtypeStruct((NC, N_PAD, D), jnp.float32),
        scratch_types=[
            pltpu.VMEM((NB, K), jnp.int32),
            pltpu.VMEM((ZB, D), jnp.float32),
            pltpu.VMEM((K, D), jnp.float32),
            pltpu.SemaphoreType.DMA,
            pltpu.VMEM_SHARED((N_PAD, D), jnp.float32),
        ],
    )
    def k(dst_hbm, out_hbm, idx_v, zb_v, ones_v, ssem, acc_sh):
        cid = lax.axis_index("c")
        sid = lax.axis_index("s")
        wid = sid * NC + cid

        @pl.loop(0, ZB)
        def _(r):
            @pl.loop(0, D, step=16)
            def _(c):
                zb_v[r, pl.ds(c, 16)] = jnp.zeros((16,), jnp.float32)

        @pl.loop(0, K)
        def _(r):
            @pl.loop(0, D, step=16)
            def _(c):
                ones_v[r, pl.ds(c, 16)] = jnp.ones((16,), jnp.float32)

        @pl.loop(0, RPS // ZB)
        def _(t):
            pltpu.sync_copy(zb_v, acc_sh.at[pl.ds(sid * RPS + t * ZB, ZB)])

        pltpu.sync_copy(dst_hbm.at[wid], idx_v)
        plsc.subcore_barrier()

        @pl.loop(0, NB)
        def _(j):
            pltpu.async_copy(ones_v, acc_sh.at[idx_v.at[j]], ssem, add=True)

        @pl.loop(0, NB)
        def _(j):
            pltpu.make_async_copy(ones_v, acc_sh.at[idx_v.at[j]], ssem).wait()

        plsc.subcore_barrier()
        pltpu.sync_copy(
            acc_sh.at[pl.ds(sid * RPS, RPS)],
            out_hbm.at[cid, pl.ds(sid * RPS, RPS)],
        )

    return k(dst3)


def _sc_agg(hp, src2, dst3):
    @functools.partial(
        pl.kernel,
        mesh=_mesh,
        out_type=jax.ShapeDtypeStruct((NC, N_PAD, D), jnp.float32),
        scratch_types=[
            pltpu.VMEM((PER_W,), jnp.int32),
            pltpu.VMEM((1, K), jnp.int32),
            pltpu.VMEM((1, K), jnp.int32),
            pltpu.VMEM((1, K), jnp.int32),
            pltpu.VMEM((K, D), jnp.float32),
            pltpu.VMEM((K, D), jnp.float32),
            pltpu.VMEM((K, D), jnp.float32),
            pltpu.VMEM((ZB, D), jnp.float32),
            pltpu.SemaphoreType.DMA,
            pltpu.SemaphoreType.DMA,
            pltpu.SemaphoreType.DMA,
            pltpu.SemaphoreType.DMA,
            pltpu.SemaphoreType.DMA,
            pltpu.SemaphoreType.DMA,
            pltpu.VMEM_SHARED((N_PAD, D), jnp.float32),
        ],
    )
    def k(hp_hbm, src_hbm, dst_hbm, out_hbm,
          src_v, dst0, dst1, dst2, rows0, rows1, rows2, zb_v,
          gsem0, gsem1, gsem2, dsem0, dsem1, dsem2, acc_sh):
        cid = lax.axis_index("c")
        sid = lax.axis_index("s")
        wid = sid * NC + cid

        dsts = (dst0, dst1, dst2)
        rows = (rows0, rows1, rows2)
        gsems = (gsem0, gsem1, gsem2)
        dsems = (dsem0, dsem1, dsem2)

        @pl.loop(0, ZB)
        def _(r):
            @pl.loop(0, D, step=16)
            def _(c):
                zb_v[r, pl.ds(c, 16)] = jnp.zeros((16,), jnp.float32)

        @pl.loop(0, RPS // ZB)
        def _(t):
            pltpu.sync_copy(zb_v, acc_sh.at[pl.ds(sid * RPS + t * ZB, ZB)])

        pltpu.sync_copy(src_hbm.at[wid], src_v)
        plsc.subcore_barrier()

        def gidx(j):
            return src_v.at[pl.ds(j * K, K)]

        def dblk(j):
            return dst_hbm.at[wid, pl.ds(j, 1)]

        def issue(j, b):
            pltpu.async_copy(hp_hbm.at[gidx(j)], rows[b], gsems[b])
            pltpu.async_copy(dblk(j), dsts[b], dsems[b])

        def consume(j, b):
            pltpu.make_async_copy(hp_hbm.at[gidx(j)], rows[b], gsems[b]).wait()
            pltpu.make_async_copy(dblk(j), dsts[b], dsems[b]).wait()
            pltpu.sync_copy(rows[b], acc_sh.at[dsts[b].at[0]], add=True)

        issue(0, 0)
        issue(1, 1)

        @pl.loop(0, NB - 2, step=3)
        def _(j):
            pltpu.make_async_copy(hp_hbm.at[gidx(j)], rows0, gsem0).wait()
            pltpu.async_copy(hp_hbm.at[gidx(j + 2)], rows2, gsem2)
            pltpu.make_async_copy(dblk(j), dst0, dsem0).wait()
            pltpu.async_copy(dblk(j + 2), dst2, dsem2)
            pltpu.sync_copy(rows0, acc_sh.at[dst0.at[0]], add=True)

            pltpu.make_async_copy(hp_hbm.at[gidx(j + 1)], rows1, gsem1).wait()
            pltpu.async_copy(hp_hbm.at[gidx(j + 3)], rows0, gsem0)
            pltpu.make_async_copy(dblk(j + 1), dst1, dsem1).wait()
            pltpu.async_copy(dblk(j + 3), dst0, dsem0)
            pltpu.sync_copy(rows1, acc_sh.at[dst1.at[0]], add=True)

            pltpu.make_async_copy(hp_hbm.at[gidx(j + 2)], rows2, gsem2).wait()
            pltpu.async_copy(hp_hbm.at[gidx(j + 4)], rows1, gsem1)
            pltpu.make_async_copy(dblk(j + 2), dst2, dsem2).wait()
            pltpu.async_copy(dblk(j + 4), dst1, dsem1)
            pltpu.sync_copy(rows2, acc_sh.at[dst2.at[0]], add=True)

        consume(NB - 2, 0)
        consume(NB - 1, 1)

        plsc.subcore_barrier()
        pltpu.sync_copy(
            acc_sh.at[pl.ds(sid * RPS, RPS)],
            out_hbm.at[cid, pl.ds(sid * RPS, RPS)],
        )

    return k(hp, src2, dst3)


_R = 1000


def _mm_body(x_ref, w_ref, o_ref):
    o_ref[...] = jnp.dot(x_ref[...], w_ref[...], preferred_element_type=jnp.float32)


def _tc_mm(x, w):
    return pl.pallas_call(
        _mm_body,
        grid=(N // _R,),
        in_specs=[
            pl.BlockSpec((_R, x.shape[1]), lambda i: (i, 0)),
            pl.BlockSpec(w.shape, lambda i: (0, 0)),
        ],
        out_specs=pl.BlockSpec((_R, w.shape[1]), lambda i: (i, 0)),
        out_shape=jax.ShapeDtypeStruct((N, w.shape[1]), jnp.float32),
    )(x, w)


def _scale_body(degp_ref, h_ref, hp_ref, dinv_ref):
    d = degp_ref[0, :, 0:1] + degp_ref[1, :, 0:1] + 1.0
    dinv = lax.rsqrt(d)
    dinv_ref[...] = dinv
    hp_ref[...] = h_ref[...] * dinv


def _tc_scale(degp, h):
    return pl.pallas_call(
        _scale_body,
        grid=(N // _R,),
        in_specs=[
            pl.BlockSpec((NC, _R, D), lambda i: (0, i, 0)),
            pl.BlockSpec((_R, D), lambda i: (i, 0)),
        ],
        out_specs=[
            pl.BlockSpec((_R, D), lambda i: (i, 0)),
            pl.BlockSpec((_R, 1), lambda i: (i, 0)),
        ],
        out_shape=[
            jax.ShapeDtypeStruct((N, D), jnp.float32),
            jax.ShapeDtypeStruct((N, 1), jnp.float32),
        ],
    )(degp, h)


def _mid_body(acc_ref, hp_ref, dinv_ref, b_ref, w_ref, o_ref):
    dinv = dinv_ref[...]
    z = dinv * (acc_ref[0] + acc_ref[1] + hp_ref[...]) + b_ref[...]
    z = jnp.maximum(z, 0.0)
    o_ref[...] = jnp.dot(z, w_ref[...], preferred_element_type=jnp.float32) * dinv


def _tc_mid(acc, hp, dinv, b, w):
    return pl.pallas_call(
        _mid_body,
        grid=(N // _R,),
        in_specs=[
            pl.BlockSpec((NC, _R, D), lambda i: (0, i, 0)),
            pl.BlockSpec((_R, D), lambda i: (i, 0)),
            pl.BlockSpec((_R, 1), lambda i: (i, 0)),
            pl.BlockSpec((1, D), lambda i: (0, 0)),
            pl.BlockSpec((D, D), lambda i: (0, 0)),
        ],
        out_specs=pl.BlockSpec((_R, D), lambda i: (i, 0)),
        out_shape=jax.ShapeDtypeStruct((N, D), jnp.float32),
    )(acc, hp, dinv, b, w)


def _final_body(acc_ref, hp_ref, dinv_ref, b_ref, wo_ref, bo_ref, emb_ref, out_ref):
    emb = dinv_ref[...] * (acc_ref[0] + acc_ref[1] + hp_ref[...]) + b_ref[...]
    emb_ref[...] = emb
    out_ref[...] = (
        jnp.dot(emb, wo_ref[...], preferred_element_type=jnp.float32) + bo_ref[...]
    )


def _tc_final(acc, hp, dinv, b, wo_t, bo):
    d_out = wo_t.shape[1]
    return pl.pallas_call(
        _final_body,
        grid=(N // _R,),
        in_specs=[
            pl.BlockSpec((NC, _R, D), lambda i: (0, i, 0)),
            pl.BlockSpec((_R, D), lambda i: (i, 0)),
            pl.BlockSpec((_R, 1), lambda i: (i, 0)),
            pl.BlockSpec((1, D), lambda i: (0, 0)),
            pl.BlockSpec((D, d_out), lambda i: (0, 0)),
            pl.BlockSpec((1, d_out), lambda i: (0, 0)),
        ],
        out_specs=[
            pl.BlockSpec((_R, D), lambda i: (i, 0)),
            pl.BlockSpec((_R, d_out), lambda i: (i, 0)),
        ],
        out_shape=[
            jax.ShapeDtypeStruct((N, D), jnp.float32),
            jax.ShapeDtypeStruct((N, d_out), jnp.float32),
        ],
    )(acc, hp, dinv, b, wo_t, bo)


@jax.jit
def kernel(x, edge_index, W1, b1, W2, b2, W_out, b_out):
    src2 = edge_index[0].reshape(NW, PER_W)
    dst3 = edge_index[1].reshape(NW, NB, K)

    degp = _sc_deg(dst3)
    h1 = _tc_mm(x, W1)
    h1p, dinv = _tc_scale(degp, h1)

    acc1 = _sc_agg(h1p, src2, dst3)
    h2p = _tc_mid(acc1, h1p, dinv, b1.reshape(1, D), W2)

    acc2 = _sc_agg(h2p, src2, dst3)
    emb, out = _tc_final(
        acc2, h2p, dinv, b2.reshape(1, D), W_out.T, b_out.reshape(1, -1)
    )
    return (out, emb)


# speedup vs baseline: 30.3506x; 1.0258x over previous
import functools

import jax
import jax.numpy as jnp
from jax import lax
from jax.experimental import pallas as pl
from jax.experimental.pallas import tpu as pltpu
from jax.experimental.pallas import tpu_sc as plsc

N = 10000
N_PAD = 10240
E = 320000
D = 128

NC = 2
NS = 16
NW = NC * NS
PER_W = E // NW
K = 80
NB = PER_W // K
RPS = N_PAD // NS
ZB = 32

_mesh = plsc.VectorSubcoreMesh(core_axis_name="c", subcore_axis_name="s")


def _sc_deg(dst3):
    @functools.partial(
        pl.kernel,
        mesh=_mesh,
        out_type=jax.ShapeDtypeStruct((NC, N_PAD, D), jnp.float32),
        scratch_types=[
            pltpu.VMEM((NB, K), jnp.int32),
            pltpu.VMEM((ZB, D), jnp.float32),
            pltpu.VMEM((K, D), jnp.float32),
            pltpu.SemaphoreType.DMA,
            pltpu.VMEM_SHARED((N_PAD, D), jnp.float32),
        ],
    )
    def k(dst_hbm, out_hbm, idx_v, zb_v, ones_v, ssem, acc_sh):
        cid = lax.axis_index("c")
        sid = lax.axis_index("s")
        wid = sid * NC + cid

        @pl.loop(0, ZB)
        def _(r):
            @pl.loop(0, D, step=16)
            def _(c):
                zb_v[r, pl.ds(c, 16)] = jnp.zeros((16,), jnp.float32)

        @pl.loop(0, K)
        def _(r):
            @pl.loop(0, D, step=16)
            def _(c):
                ones_v[r, pl.ds(c, 16)] = jnp.ones((16,), jnp.float32)

        @pl.loop(0, RPS // ZB)
        def _(t):
            pltpu.sync_copy(zb_v, acc_sh.at[pl.ds(sid * RPS + t * ZB, ZB)])

        pltpu.sync_copy(dst_hbm.at[wid], idx_v)
        plsc.subcore_barrier()

        @pl.loop(0, NB)
        def _(j):
            pltpu.async_copy(ones_v, acc_sh.at[idx_v.at[j]], ssem, add=True)

        @pl.loop(0, NB)
        def _(j):
            pltpu.make_async_copy(ones_v, acc_sh.at[idx_v.at[j]], ssem).wait()

        plsc.subcore_barrier()
        pltpu.sync_copy(
            acc_sh.at[pl.ds(sid * RPS, RPS)],
            out_hbm.at[cid, pl.ds(sid * RPS, RPS)],
        )

    return k(dst3)


def _sc_agg(hp, src2, dst3):
    @functools.partial(
        pl.kernel,
        mesh=_mesh,
        out_type=jax.ShapeDtypeStruct((NC, N_PAD, D), jnp.float32),
        scratch_types=[
            pltpu.VMEM((1, K), jnp.int32),
            pltpu.VMEM((1, K), jnp.int32),
            pltpu.VMEM((1, K), jnp.int32),
            pltpu.VMEM((1, K), jnp.int32),
            pltpu.VMEM((1, K), jnp.int32),
            pltpu.VMEM((1, K), jnp.int32),
            pltpu.VMEM((1, K), jnp.int32),
            pltpu.VMEM((1, K), jnp.int32),
            pltpu.VMEM((K, D), jnp.float32),
            pltpu.VMEM((K, D), jnp.float32),
            pltpu.VMEM((K, D), jnp.float32),
            pltpu.VMEM((K, D), jnp.float32),
            pltpu.VMEM((ZB, D), jnp.float32),
            pltpu.SemaphoreType.DMA,
            pltpu.SemaphoreType.DMA,
            pltpu.SemaphoreType.DMA,
            pltpu.SemaphoreType.DMA,
            pltpu.SemaphoreType.DMA,
            pltpu.SemaphoreType.DMA,
            pltpu.SemaphoreType.DMA,
            pltpu.SemaphoreType.DMA,
            pltpu.SemaphoreType.DMA,
            pltpu.SemaphoreType.DMA,
            pltpu.SemaphoreType.DMA,
            pltpu.SemaphoreType.DMA,
            pltpu.VMEM_SHARED((N_PAD, D), jnp.float32),
        ],
    )
    def k(hp_hbm, src_hbm, dst_hbm, out_hbm,
          src0, src1, src2_, src3, dst0, dst1, dst2, dst3_,
          rows0, rows1, rows2, rows3, zb_v,
          ssem0, ssem1, ssem2, ssem3, gsem0, gsem1, gsem2, gsem3,
          dsem0, dsem1, dsem2, dsem3, acc_sh):
        cid = lax.axis_index("c")
        sid = lax.axis_index("s")
        wid = sid * NC + cid

        srcs = (src0, src1, src2_, src3)
        dsts = (dst0, dst1, dst2, dst3_)
        rows = (rows0, rows1, rows2, rows3)
        ssems = (ssem0, ssem1, ssem2, ssem3)
        gsems = (gsem0, gsem1, gsem2, gsem3)
        dsems = (dsem0, dsem1, dsem2, dsem3)

        @pl.loop(0, ZB)
        def _(r):
            @pl.loop(0, D, step=16)
            def _(c):
                zb_v[r, pl.ds(c, 16)] = jnp.zeros((16,), jnp.float32)

        @pl.loop(0, RPS // ZB)
        def _(t):
            pltpu.sync_copy(zb_v, acc_sh.at[pl.ds(sid * RPS + t * ZB, ZB)])

        plsc.subcore_barrier()

        def sblk(j):
            return src_hbm.at[wid, pl.ds(j, 1)]

        def dblk(j):
            return dst_hbm.at[wid, pl.ds(j, 1)]

        def gat(b):
            return pltpu.make_async_copy(
                hp_hbm.at[srcs[b].at[0]], rows[b], gsems[b]
            )

        def block(j, b, nxt):
            b3 = (b + 3) % 4
            if nxt >= 1:
                pltpu.make_async_copy(sblk(j + 3), srcs[b3], ssems[b3]).wait()
                gat(b3).start()
            gat(b).wait()
            if nxt >= 2:
                pltpu.async_copy(sblk(j + 4), srcs[b], ssems[b])
            pltpu.make_async_copy(dblk(j), dsts[b], dsems[b]).wait()
            pltpu.sync_copy(rows[b], acc_sh.at[dsts[b].at[0]], add=True)
            if nxt >= 2:
                pltpu.async_copy(dblk(j + 4), dsts[b], dsems[b])

        for b in range(4):
            pltpu.async_copy(sblk(b), srcs[b], ssems[b])
            pltpu.async_copy(dblk(b), dsts[b], dsems[b])
        for b in range(3):
            pltpu.make_async_copy(sblk(b), srcs[b], ssems[b]).wait()
            gat(b).start()

        @pl.loop(0, NB - 5, step=4)
        def _(j):
            block(j + 0, 0, 2)
            block(j + 1, 1, 2)
            block(j + 2, 2, 2)
            block(j + 3, 3, 2)

        block(NB - 5, 0, 2)
        block(NB - 4, 1, 1)
        block(NB - 3, 2, 0)
        block(NB - 2, 3, 0)
        block(NB - 1, 0, 0)

        plsc.subcore_barrier()
        pltpu.sync_copy(
            acc_sh.at[pl.ds(sid * RPS, RPS)],
            out_hbm.at[cid, pl.ds(sid * RPS, RPS)],
        )

    return k(hp, src2, dst3)


_R = 1000


def _mm_body(x_ref, w_ref, o_ref):
    o_ref[...] = jnp.dot(x_ref[...], w_ref[...], preferred_element_type=jnp.float32)


def _tc_mm(x, w):
    return pl.pallas_call(
        _mm_body,
        grid=(N // _R,),
        in_specs=[
            pl.BlockSpec((_R, x.shape[1]), lambda i: (i, 0)),
            pl.BlockSpec(w.shape, lambda i: (0, 0)),
        ],
        out_specs=pl.BlockSpec((_R, w.shape[1]), lambda i: (i, 0)),
        out_shape=jax.ShapeDtypeStruct((N, w.shape[1]), jnp.float32),
    )(x, w)


def _scale_body(degp_ref, h_ref, hp_ref, dinv_ref):
    d = degp_ref[0, :, 0:1] + degp_ref[1, :, 0:1] + 1.0
    dinv = lax.rsqrt(d)
    dinv_ref[...] = dinv
    hp_ref[...] = h_ref[...] * dinv


def _tc_scale(degp, h):
    return pl.pallas_call(
        _scale_body,
        grid=(N // _R,),
        in_specs=[
            pl.BlockSpec((NC, _R, D), lambda i: (0, i, 0)),
            pl.BlockSpec((_R, D), lambda i: (i, 0)),
        ],
        out_specs=[
            pl.BlockSpec((_R, D), lambda i: (i, 0)),
            pl.BlockSpec((_R, 1), lambda i: (i, 0)),
        ],
        out_shape=[
            jax.ShapeDtypeStruct((N, D), jnp.float32),
            jax.ShapeDtypeStruct((N, 1), jnp.float32),
        ],
    )(degp, h)


def _mid_body(acc_ref, hp_ref, dinv_ref, b_ref, w_ref, o_ref):
    dinv = dinv_ref[...]
    z = dinv * (acc_ref[0] + acc_ref[1] + hp_ref[...]) + b_ref[...]
    z = jnp.maximum(z, 0.0)
    o_ref[...] = jnp.dot(z, w_ref[...], preferred_element_type=jnp.float32) * dinv


def _tc_mid(acc, hp, dinv, b, w):
    return pl.pallas_call(
        _mid_body,
        grid=(N // _R,),
        in_specs=[
            pl.BlockSpec((NC, _R, D), lambda i: (0, i, 0)),
            pl.BlockSpec((_R, D), lambda i: (i, 0)),
            pl.BlockSpec((_R, 1), lambda i: (i, 0)),
            pl.BlockSpec((1, D), lambda i: (0, 0)),
            pl.BlockSpec((D, D), lambda i: (0, 0)),
        ],
        out_specs=pl.BlockSpec((_R, D), lambda i: (i, 0)),
        out_shape=jax.ShapeDtypeStruct((N, D), jnp.float32),
    )(acc, hp, dinv, b, w)


def _final_body(acc_ref, hp_ref, dinv_ref, b_ref, wo_ref, bo_ref, emb_ref, out_ref):
    emb = dinv_ref[...] * (acc_ref[0] + acc_ref[1] + hp_ref[...]) + b_ref[...]
    emb_ref[...] = emb
    out_ref[...] = (
        jnp.dot(emb, wo_ref[...], preferred_element_type=jnp.float32) + bo_ref[...]
    )


def _tc_final(acc, hp, dinv, b, wo_t, bo):
    d_out = wo_t.shape[1]
    return pl.pallas_call(
        _final_body,
        grid=(N // _R,),
        in_specs=[
            pl.BlockSpec((NC, _R, D), lambda i: (0, i, 0)),
            pl.BlockSpec((_R, D), lambda i: (i, 0)),
            pl.BlockSpec((_R, 1), lambda i: (i, 0)),
            pl.BlockSpec((1, D), lambda i: (0, 0)),
            pl.BlockSpec((D, d_out), lambda i: (0, 0)),
            pl.BlockSpec((1, d_out), lambda i: (0, 0)),
        ],
        out_specs=[
            pl.BlockSpec((_R, D), lambda i: (i, 0)),
            pl.BlockSpec((_R, d_out), lambda i: (i, 0)),
        ],
        out_shape=[
            jax.ShapeDtypeStruct((N, D), jnp.float32),
            jax.ShapeDtypeStruct((N, d_out), jnp.float32),
        ],
    )(acc, hp, dinv, b, wo_t, bo)


@jax.jit
def kernel(x, edge_index, W1, b1, W2, b2, W_out, b_out):
    src3 = edge_index[0].reshape(NW, NB, K)
    dst3 = edge_index[1].reshape(NW, NB, K)

    degp = _sc_deg(dst3)
    h1 = _tc_mm(x, W1)
    h1p, dinv = _tc_scale(degp, h1)

    acc1 = _sc_agg(h1p, src3, dst3)
    h2p = _tc_mid(acc1, h1p, dinv, b1.reshape(1, D), W2)

    acc2 = _sc_agg(h2p, src3, dst3)
    emb, out = _tc_final(
        acc2, h2p, dinv, b2.reshape(1, D), W_out.T, b_out.reshape(1, -1)
    )
    return (out, emb)


# speedup vs baseline: 30.3796x; 1.0010x over previous
import functools

import jax
import jax.numpy as jnp
from jax import lax
from jax.experimental import pallas as pl
from jax.experimental.pallas import tpu as pltpu
from jax.experimental.pallas import tpu_sc as plsc

N = 10000
N_PAD = 10240
E = 320000
D = 128

NC = 2
NS = 16
NW = NC * NS
PER_W = E // NW
K = 80
NB = PER_W // K
RPS = N_PAD // NS
ZB = 32

_mesh = plsc.VectorSubcoreMesh(core_axis_name="c", subcore_axis_name="s")


def _sc_deg(dst3):
    @functools.partial(
        pl.kernel,
        mesh=_mesh,
        out_type=jax.ShapeDtypeStruct((NC, N_PAD, D), jnp.float32),
        scratch_types=[
            pltpu.VMEM((NB, K), jnp.int32),
            pltpu.VMEM((ZB, D), jnp.float32),
            pltpu.VMEM((K, D), jnp.float32),
            pltpu.SemaphoreType.DMA,
            pltpu.VMEM_SHARED((N_PAD, D), jnp.float32),
        ],
    )
    def k(dst_hbm, out_hbm, idx_v, zb_v, ones_v, ssem, acc_sh):
        cid = lax.axis_index("c")
        sid = lax.axis_index("s")
        wid = sid * NC + cid

        @pl.loop(0, ZB)
        def _(r):
            @pl.loop(0, D, step=16)
            def _(c):
                zb_v[r, pl.ds(c, 16)] = jnp.zeros((16,), jnp.float32)

        @pl.loop(0, K)
        def _(r):
            @pl.loop(0, D, step=16)
            def _(c):
                ones_v[r, pl.ds(c, 16)] = jnp.ones((16,), jnp.float32)

        @pl.loop(0, RPS // ZB)
        def _(t):
            pltpu.sync_copy(zb_v, acc_sh.at[pl.ds(sid * RPS + t * ZB, ZB)])

        pltpu.sync_copy(dst_hbm.at[wid], idx_v)
        plsc.subcore_barrier()

        @pl.loop(0, NB)
        def _(j):
            pltpu.async_copy(ones_v, acc_sh.at[idx_v.at[j]], ssem, add=True)

        @pl.loop(0, NB)
        def _(j):
            pltpu.make_async_copy(ones_v, acc_sh.at[idx_v.at[j]], ssem).wait()

        plsc.subcore_barrier()
        pltpu.sync_copy(
            acc_sh.at[pl.ds(sid * RPS, RPS)],
            out_hbm.at[cid, pl.ds(sid * RPS, RPS)],
        )

    return k(dst3)


def _sc_agg(hp, src2, dst3):
    @functools.partial(
        pl.kernel,
        mesh=_mesh,
        out_type=jax.ShapeDtypeStruct((NC, N_PAD, D), jnp.float32),
        scratch_types=[
            pltpu.VMEM((1, K), jnp.int32),
            pltpu.VMEM((1, K), jnp.int32),
            pltpu.VMEM((1, K), jnp.int32),
            pltpu.VMEM((1, K), jnp.int32),
            pltpu.VMEM((1, K), jnp.int32),
            pltpu.VMEM((1, K), jnp.int32),
            pltpu.VMEM((1, K), jnp.int32),
            pltpu.VMEM((1, K), jnp.int32),
            pltpu.VMEM((K, D), jnp.float32),
            pltpu.VMEM((K, D), jnp.float32),
            pltpu.VMEM((K, D), jnp.float32),
            pltpu.VMEM((K, D), jnp.float32),
            pltpu.VMEM((ZB, D), jnp.float32),
            pltpu.SemaphoreType.DMA,
            pltpu.SemaphoreType.DMA,
            pltpu.SemaphoreType.DMA,
            pltpu.SemaphoreType.DMA,
            pltpu.SemaphoreType.DMA,
            pltpu.SemaphoreType.DMA,
            pltpu.SemaphoreType.DMA,
            pltpu.SemaphoreType.DMA,
            pltpu.SemaphoreType.DMA,
            pltpu.SemaphoreType.DMA,
            pltpu.SemaphoreType.DMA,
            pltpu.SemaphoreType.DMA,
            pltpu.VMEM_SHARED((N_PAD, D), jnp.float32),
        ],
    )
    def k(hp_hbm, src_hbm, dst_hbm, out_hbm,
          src0, src1, src2_, src3, dst0, dst1, dst2, dst3_,
          rows0, rows1, rows2, rows3, zb_v,
          ssem0, ssem1, ssem2, ssem3, gsem0, gsem1, gsem2, gsem3,
          dsem0, dsem1, dsem2, dsem3, acc_sh):
        cid = lax.axis_index("c")
        sid = lax.axis_index("s")
        wid = sid * NC + cid

        srcs = (src0, src1, src2_, src3)
        dsts = (dst0, dst1, dst2, dst3_)
        rows = (rows0, rows1, rows2, rows3)
        ssems = (ssem0, ssem1, ssem2, ssem3)
        gsems = (gsem0, gsem1, gsem2, gsem3)
        dsems = (dsem0, dsem1, dsem2, dsem3)

        @pl.loop(0, ZB)
        def _(r):
            @pl.loop(0, D, step=16)
            def _(c):
                zb_v[r, pl.ds(c, 16)] = jnp.zeros((16,), jnp.float32)

        @pl.loop(0, RPS // ZB)
        def _(t):
            pltpu.sync_copy(zb_v, acc_sh.at[pl.ds(sid * RPS + t * ZB, ZB)])

        plsc.subcore_barrier()

        def sblk(j):
            return src_hbm.at[wid, pl.ds(j, 1)]

        def dblk(j):
            return dst_hbm.at[wid, pl.ds(j, 1)]

        def gat(b):
            return pltpu.make_async_copy(
                hp_hbm.at[srcs[b].at[0]], rows[b], gsems[b]
            )

        def block(j, b, nxt):
            b3 = (b + 3) % 4
            if nxt >= 1:
                pltpu.make_async_copy(sblk(j + 3), srcs[b3], ssems[b3]).wait()
                gat(b3).start()
            gat(b).wait()
            if nxt >= 2:
                pltpu.async_copy(sblk(j + 4), srcs[b], ssems[b])
            pltpu.make_async_copy(dblk(j), dsts[b], dsems[b]).wait()
            pltpu.sync_copy(rows[b], acc_sh.at[dsts[b].at[0]], add=True)
            if nxt >= 2:
                pltpu.async_copy(dblk(j + 4), dsts[b], dsems[b])

        for b in range(4):
            pltpu.async_copy(sblk(b), srcs[b], ssems[b])
            pltpu.async_copy(dblk(b), dsts[b], dsems[b])
        for b in range(3):
            pltpu.make_async_copy(sblk(b), srcs[b], ssems[b]).wait()
            gat(b).start()

        @pl.loop(0, NB - 5, step=4)
        def _(j):
            block(j + 0, 0, 2)
            block(j + 1, 1, 2)
            block(j + 2, 2, 2)
            block(j + 3, 3, 2)

        block(NB - 5, 0, 2)
        block(NB - 4, 1, 1)
        block(NB - 3, 2, 0)
        block(NB - 2, 3, 0)
        block(NB - 1, 0, 0)

        plsc.subcore_barrier()
        pltpu.sync_copy(
            acc_sh.at[pl.ds(sid * RPS, RPS)],
            out_hbm.at[cid, pl.ds(sid * RPS, RPS)],
        )

    return k(hp, src2, dst3)


_R = 1000


def _scale_body(degp_ref, x_ref, w_ref, hp_ref, dinv_ref):
    d = degp_ref[0, :, 0:1] + degp_ref[1, :, 0:1] + 1.0
    dinv = lax.rsqrt(d)
    dinv_ref[...] = dinv
    h = jnp.dot(x_ref[...], w_ref[...], preferred_element_type=jnp.float32)
    hp_ref[...] = h * dinv


def _tc_scale(degp, x, w):
    return pl.pallas_call(
        _scale_body,
        grid=(N // _R,),
        in_specs=[
            pl.BlockSpec((NC, _R, D), lambda i: (0, i, 0)),
            pl.BlockSpec((_R, D), lambda i: (i, 0)),
            pl.BlockSpec(w.shape, lambda i: (0, 0)),
        ],
        out_specs=[
            pl.BlockSpec((_R, D), lambda i: (i, 0)),
            pl.BlockSpec((_R, 1), lambda i: (i, 0)),
        ],
        out_shape=[
            jax.ShapeDtypeStruct((N, D), jnp.float32),
            jax.ShapeDtypeStruct((N, 1), jnp.float32),
        ],
    )(degp, x, w)


def _mid_body(acc_ref, hp_ref, dinv_ref, b_ref, w_ref, o_ref):
    dinv = dinv_ref[...]
    z = dinv * (acc_ref[0] + acc_ref[1] + hp_ref[...]) + b_ref[...]
    z = jnp.maximum(z, 0.0)
    o_ref[...] = jnp.dot(z, w_ref[...], preferred_element_type=jnp.float32) * dinv


def _tc_mid(acc, hp, dinv, b, w):
    return pl.pallas_call(
        _mid_body,
        grid=(N // _R,),
        in_specs=[
            pl.BlockSpec((NC, _R, D), lambda i: (0, i, 0)),
            pl.BlockSpec((_R, D), lambda i: (i, 0)),
            pl.BlockSpec((_R, 1), lambda i: (i, 0)),
            pl.BlockSpec((1, D), lambda i: (0, 0)),
            pl.BlockSpec((D, D), lambda i: (0, 0)),
        ],
        out_specs=pl.BlockSpec((_R, D), lambda i: (i, 0)),
        out_shape=jax.ShapeDtypeStruct((N, D), jnp.float32),
    )(acc, hp, dinv, b, w)


def _final_body(acc_ref, hp_ref, dinv_ref, b_ref, wo_ref, bo_ref, emb_ref, out_ref):
    emb = dinv_ref[...] * (acc_ref[0] + acc_ref[1] + hp_ref[...]) + b_ref[...]
    emb_ref[...] = emb
    out_ref[...] = (
        jnp.dot(emb, wo_ref[...], preferred_element_type=jnp.float32) + bo_ref[...]
    )


def _tc_final(acc, hp, dinv, b, wo_t, bo):
    d_out = wo_t.shape[1]
    return pl.pallas_call(
        _final_body,
        grid=(N // _R,),
        in_specs=[
            pl.BlockSpec((NC, _R, D), lambda i: (0, i, 0)),
            pl.BlockSpec((_R, D), lambda i: (i, 0)),
            pl.BlockSpec((_R, 1), lambda i: (i, 0)),
            pl.BlockSpec((1, D), lambda i: (0, 0)),
            pl.BlockSpec((D, d_out), lambda i: (0, 0)),
            pl.BlockSpec((1, d_out), lambda i: (0, 0)),
        ],
        out_specs=[
            pl.BlockSpec((_R, D), lambda i: (i, 0)),
            pl.BlockSpec((_R, d_out), lambda i: (i, 0)),
        ],
        out_shape=[
            jax.ShapeDtypeStruct((N, D), jnp.float32),
            jax.ShapeDtypeStruct((N, d_out), jnp.float32),
        ],
    )(acc, hp, dinv, b, wo_t, bo)


@jax.jit
def kernel(x, edge_index, W1, b1, W2, b2, W_out, b_out):
    src3 = edge_index[0].reshape(NW, NB, K)
    dst3 = edge_index[1].reshape(NW, NB, K)

    degp = _sc_deg(dst3)
    h1p, dinv = _tc_scale(degp, x, W1)

    acc1 = _sc_agg(h1p, src3, dst3)
    h2p = _tc_mid(acc1, h1p, dinv, b1.reshape(1, D), W2)

    acc2 = _sc_agg(h2p, src3, dst3)
    emb, out = _tc_final(
        acc2, h2p, dinv, b2.reshape(1, D), W_out.T, b_out.reshape(1, -1)
    )
    return (out, emb)
